# baseline (speedup 1.0000x reference)
import jax
import jax.numpy as jnp
from jax.experimental import pallas as pl


def _pe_kernel(row_ref, col_ref, out_ref):
    c = row_ref.shape[1]
    h = row_ref.shape[0]
    w = col_ref.shape[0]
    ce_t = col_ref[...].T
    re_t = row_ref[...].T
    out_ref[0:c] = jnp.broadcast_to(ce_t[:, None, :], (c, h, w))
    out_ref[c : 2 * c] = jnp.broadcast_to(re_t[:, :, None], (c, h, w))


def kernel(x, row_embed, col_embed):
    h, w = x.shape[-2], x.shape[-1]
    c = row_embed.shape[1]
    out = pl.pallas_call(
        _pe_kernel,
        out_shape=jax.ShapeDtypeStruct((2 * c, h, w), jnp.float32),
    )(row_embed[:h], col_embed[:w])
    return out
